# Initial kernel scaffold; baseline (speedup 1.0000x reference)
import jax
import jax.numpy as jnp
from jax.experimental import pallas as pl


def kernel(x, params):
    raise NotImplementedError("write your pallas kernel here")


# speedup vs baseline: 1.9020x; 1.9020x over previous
import functools

import jax
import jax.numpy as jnp
from jax import lax
from jax.experimental import pallas as pl
from jax.experimental.pallas import tpu as pltpu
from jax.experimental.pallas import tpu_sc as plsc

EMBED = 384
HEADS = 12
DH = EMBED // HEADS
HID = 1536
NUM_EXPERT = 8
TOP_K = 2
PATCH = 16
GRID = 14
NUM_PATCHES = GRID * GRID
NUM_CLASSES = 1000
BATCH = 16
T = NUM_PATCHES + 1
TP = 200
N = BATCH * TP
TM = 320
NA = N * TOP_K
TILE = 256
NTILES = NA // TILE + NUM_EXPERT
MPAD = NTILES * TILE
NEG = -1e30


def _mm(a, b):
    return lax.dot_general(
        a.astype(jnp.bfloat16), b.astype(jnp.bfloat16),
        (((a.ndim - 1,), (0,)), ((), ())),
        preferred_element_type=jnp.float32)


def _mm_hi(a, b):
    return lax.dot_general(
        a, b, (((a.ndim - 1,), (0,)), ((), ())),
        precision=lax.Precision.HIGHEST, preferred_element_type=jnp.float32)


def _ln(x, w, b):
    mu = jnp.mean(x, axis=-1, keepdims=True)
    xc = x - mu
    var = jnp.mean(xc * xc, axis=-1, keepdims=True)
    return xc * lax.rsqrt(var + 1e-5) * w + b


def _gelu(x):
    return 0.5 * x * (1.0 + lax.erf(x * 0.7071067811865476))


def _embed_body(p_ref, w_ref, b_ref, o_ref):
    o_ref[...] = _mm(p_ref[...], w_ref[...]) + b_ref[...]


def _embed(patches, w, b):
    M = patches.shape[0]
    tm = 392
    return pl.pallas_call(
        _embed_body,
        grid=(M // tm,),
        in_specs=[
            pl.BlockSpec((tm, 3 * PATCH * PATCH), lambda i: (i, 0)),
            pl.BlockSpec((3 * PATCH * PATCH, EMBED), lambda i: (0, 0)),
            pl.BlockSpec((1, EMBED), lambda i: (0, 0)),
        ],
        out_specs=pl.BlockSpec((tm, EMBED), lambda i: (i, 0)),
        out_shape=jax.ShapeDtypeStruct((M, EMBED), jnp.float32),
    )(patches, w, b)


def _lnmm_body(x_ref, lw_ref, lb_ref, w_ref, b_ref, o_ref):
    xn = _ln(x_ref[...], lw_ref[...], lb_ref[...])
    o_ref[...] = _mm(xn, w_ref[...]) + b_ref[...]


def _ln_qkv(h, lw, lb, w, b):
    return pl.pallas_call(
        _lnmm_body,
        grid=(N // TM,),
        in_specs=[
            pl.BlockSpec((TM, EMBED), lambda i: (i, 0)),
            pl.BlockSpec((1, EMBED), lambda i: (0, 0)),
            pl.BlockSpec((1, EMBED), lambda i: (0, 0)),
            pl.BlockSpec((EMBED, 3 * EMBED), lambda i: (0, 0)),
            pl.BlockSpec((1, 3 * EMBED), lambda i: (0, 0)),
        ],
        out_specs=pl.BlockSpec((TM, 3 * EMBED), lambda i: (i, 0)),
        out_shape=jax.ShapeDtypeStruct((N, 3 * EMBED), jnp.float32),
    )(h, lw, lb, w, b)


def _attn_body(qkv_ref, o_ref):
    scale = DH ** -0.5
    qkv = qkv_ref[0]
    col = lax.broadcasted_iota(jnp.int32, (TP, TP), 1)
    mask = jnp.where(col >= T, NEG, 0.0)
    outs = []
    for h in range(HEADS):
        q = qkv[:, DH * h:DH * (h + 1)]
        k = qkv[:, EMBED + DH * h:EMBED + DH * (h + 1)]
        v = qkv[:, 2 * EMBED + DH * h:2 * EMBED + DH * (h + 1)]
        s = lax.dot_general(
            q.astype(jnp.bfloat16), k.astype(jnp.bfloat16),
            (((1,), (1,)), ((), ())),
            preferred_element_type=jnp.float32) * scale + mask
        s = s - jnp.max(s, axis=-1, keepdims=True)
        p = jnp.exp(s)
        p = p / jnp.sum(p, axis=-1, keepdims=True)
        outs.append(_mm(p, v))
    o_ref[0] = jnp.concatenate(outs, axis=-1)


def _attn(qkv):
    qkv3 = qkv.reshape(BATCH, TP, 3 * EMBED)
    out = pl.pallas_call(
        _attn_body,
        grid=(BATCH,),
        in_specs=[pl.BlockSpec((1, TP, 3 * EMBED), lambda i: (i, 0, 0))],
        out_specs=pl.BlockSpec((1, TP, EMBED), lambda i: (i, 0, 0)),
        out_shape=jax.ShapeDtypeStruct((BATCH, TP, EMBED), jnp.float32),
    )(qkv3)
    return out.reshape(N, EMBED)


def _proj_mlp_body(ao_ref, h_ref, pw_ref, pb_ref, lw_ref, lb_ref,
                   w1_ref, b1_ref, w2_ref, b2_ref, o_ref):
    h2 = h_ref[...] + _mm(ao_ref[...], pw_ref[...]) + pb_ref[...]
    xn = _ln(h2, lw_ref[...], lb_ref[...])
    hmid = _gelu(_mm(xn, w1_ref[...]) + b1_ref[...])
    o_ref[...] = h2 + _mm(hmid, w2_ref[...]) + b2_ref[...]


def _proj_mlp(ao, h, pw, pb, lw, lb, w1, b1, w2, b2):
    return pl.pallas_call(
        _proj_mlp_body,
        grid=(N // TM,),
        in_specs=[
            pl.BlockSpec((TM, EMBED), lambda i: (i, 0)),
            pl.BlockSpec((TM, EMBED), lambda i: (i, 0)),
            pl.BlockSpec((EMBED, EMBED), lambda i: (0, 0)),
            pl.BlockSpec((1, EMBED), lambda i: (0, 0)),
            pl.BlockSpec((1, EMBED), lambda i: (0, 0)),
            pl.BlockSpec((1, EMBED), lambda i: (0, 0)),
            pl.BlockSpec((EMBED, HID), lambda i: (0, 0)),
            pl.BlockSpec((1, HID), lambda i: (0, 0)),
            pl.BlockSpec((HID, EMBED), lambda i: (0, 0)),
            pl.BlockSpec((1, EMBED), lambda i: (0, 0)),
        ],
        out_specs=pl.BlockSpec((TM, EMBED), lambda i: (i, 0)),
        out_shape=jax.ShapeDtypeStruct((N, EMBED), jnp.float32),
    )(ao, h, pw, pb, lw, lb, w1, b1, w2, b2)


def _proj_gate_body(ao_ref, h_ref, pw_ref, pb_ref, lw_ref, lb_ref,
                    gw_ref, gb_ref, h2_ref, xn_ref, i01_ref, s01_ref):
    h2 = h_ref[...] + _mm(ao_ref[...], pw_ref[...]) + pb_ref[...]
    xn = _ln(h2, lw_ref[...], lb_ref[...])
    h2_ref[...] = h2
    xn_ref[...] = xn
    logits = _mm_hi(xn, gw_ref[...]) + gb_ref[...]
    iot = lax.broadcasted_iota(jnp.int32, logits.shape, 1)
    m0 = jnp.max(logits, axis=-1, keepdims=True)
    i0 = jnp.min(jnp.where(logits >= m0, iot, NUM_EXPERT), axis=-1, keepdims=True)
    l1 = jnp.where(iot == i0, NEG, logits)
    m1 = jnp.max(l1, axis=-1, keepdims=True)
    i1 = jnp.min(jnp.where(l1 >= m1, iot, NUM_EXPERT), axis=-1, keepdims=True)
    e1 = jnp.exp(m1 - m0)
    s0 = 1.0 / (1.0 + e1)
    i01_ref[...] = jnp.concatenate([i0, i1], axis=-1)
    s01_ref[...] = jnp.concatenate([s0, 1.0 - s0], axis=-1)


def _proj_gate(ao, h, pw, pb, lw, lb, gw, gb):
    return pl.pallas_call(
        _proj_gate_body,
        grid=(N // TM,),
        in_specs=[
            pl.BlockSpec((TM, EMBED), lambda i: (i, 0)),
            pl.BlockSpec((TM, EMBED), lambda i: (i, 0)),
            pl.BlockSpec((EMBED, EMBED), lambda i: (0, 0)),
            pl.BlockSpec((1, EMBED), lambda i: (0, 0)),
            pl.BlockSpec((1, EMBED), lambda i: (0, 0)),
            pl.BlockSpec((1, EMBED), lambda i: (0, 0)),
            pl.BlockSpec((EMBED, NUM_EXPERT), lambda i: (0, 0)),
            pl.BlockSpec((1, NUM_EXPERT), lambda i: (0, 0)),
        ],
        out_specs=[
            pl.BlockSpec((TM, EMBED), lambda i: (i, 0)),
            pl.BlockSpec((TM, EMBED), lambda i: (i, 0)),
            pl.BlockSpec((TM, TOP_K), lambda i: (i, 0)),
            pl.BlockSpec((TM, TOP_K), lambda i: (i, 0)),
        ],
        out_shape=[
            jax.ShapeDtypeStruct((N, EMBED), jnp.float32),
            jax.ShapeDtypeStruct((N, EMBED), jnp.float32),
            jax.ShapeDtypeStruct((N, TOP_K), jnp.int32),
            jax.ShapeDtypeStruct((N, TOP_K), jnp.float32),
        ],
    )(ao, h, pw, pb, lw, lb, gw, gb)


def _ggemm_body(eid_ref, x_ref, w1_ref, b1_ref, w2_ref, b2_ref, wt_ref, o_ref):
    x = x_ref[...]
    hmid = _gelu(_mm(x, w1_ref[0]) + b1_ref[0])
    o = _mm(hmid, w2_ref[0]) + b2_ref[0]
    o_ref[...] = o * wt_ref[...]


def _ggemm(x_sorted, w1, b1, w2, b2, wt, eid):
    grid_spec = pltpu.PrefetchScalarGridSpec(
        num_scalar_prefetch=1,
        grid=(NTILES,),
        in_specs=[
            pl.BlockSpec((TILE, EMBED), lambda g, eid: (g, 0)),
            pl.BlockSpec((1, EMBED, HID), lambda g, eid: (eid[g], 0, 0)),
            pl.BlockSpec((1, 1, HID), lambda g, eid: (eid[g], 0, 0)),
            pl.BlockSpec((1, HID, EMBED), lambda g, eid: (eid[g], 0, 0)),
            pl.BlockSpec((1, 1, EMBED), lambda g, eid: (eid[g], 0, 0)),
            pl.BlockSpec((TILE, 1), lambda g, eid: (g, 0)),
        ],
        out_specs=pl.BlockSpec((TILE, EMBED), lambda g, eid: (g, 0)),
    )
    return pl.pallas_call(
        _ggemm_body,
        grid_spec=grid_spec,
        out_shape=jax.ShapeDtypeStruct((MPAD, EMBED), jnp.float32),
    )(eid, x_sorted, w1, b1, w2, b2, wt)


def _combine_body(h2_ref, g_ref, o_ref):
    o_ref[...] = h2_ref[...] + g_ref[:, 0, :] + g_ref[:, 1, :]


def _combine(h2, g):
    g3 = g.reshape(N, TOP_K, EMBED)
    return pl.pallas_call(
        _combine_body,
        grid=(N // TM,),
        in_specs=[
            pl.BlockSpec((TM, EMBED), lambda i: (i, 0)),
            pl.BlockSpec((TM, TOP_K, EMBED), lambda i: (i, 0, 0)),
        ],
        out_specs=pl.BlockSpec((TM, EMBED), lambda i: (i, 0)),
        out_shape=jax.ShapeDtypeStruct((N, EMBED), jnp.float32),
    )(h2, g3)


def _head_body(x_ref, lw_ref, lb_ref, w_ref, b_ref, o_ref):
    xn = _ln(x_ref[...], lw_ref[...], lb_ref[...])
    o_ref[...] = _mm(xn, w_ref[...]) + b_ref[...]


def _head(hcls, lw, lb, w, b):
    return pl.pallas_call(
        _head_body,
        in_specs=[
            pl.BlockSpec((BATCH, EMBED), lambda: (0, 0)),
            pl.BlockSpec((1, EMBED), lambda: (0, 0)),
            pl.BlockSpec((1, EMBED), lambda: (0, 0)),
            pl.BlockSpec((EMBED, NUM_CLASSES), lambda: (0, 0)),
            pl.BlockSpec((1, NUM_CLASSES), lambda: (0, 0)),
        ],
        out_specs=pl.BlockSpec((BATCH, NUM_CLASSES), lambda: (0, 0)),
        out_shape=jax.ShapeDtypeStruct((BATCH, NUM_CLASSES), jnp.float32),
    )(hcls, lw, lb, w, b)


@functools.cache
def _sc_gather_fn(rows_out, table_rows):
    NW = 32
    per_w = rows_out // NW
    c0 = 8
    for c in range(8, 129, 8):
        if per_w % c == 0:
            c0 = c
    nch = per_w // c0
    mesh = plsc.VectorSubcoreMesh(core_axis_name="c", subcore_axis_name="s")

    @functools.partial(
        pl.kernel, mesh=mesh,
        out_type=jax.ShapeDtypeStruct((rows_out, EMBED), jnp.float32),
        scratch_types=[
            pltpu.VMEM((c0,), jnp.int32),
            pltpu.VMEM((c0, EMBED), jnp.float32),
            pltpu.SemaphoreType.DMA,
        ],
    )
    def k(table_hbm, idx_hbm, out_hbm, idx_v, rows_v, sem):
        wid = lax.axis_index("s") * 2 + lax.axis_index("c")
        base = wid * per_w
        for c in range(nch):
            pltpu.sync_copy(idx_hbm.at[pl.ds(base + c * c0, c0)], idx_v)
            pltpu.async_copy(table_hbm.at[idx_v], rows_v, sem).wait()
            pltpu.sync_copy(rows_v, out_hbm.at[pl.ds(base + c * c0, c0)])

    return k


def _gather_rows(table, idx):
    return _sc_gather_fn(idx.shape[0], table.shape[0])(table, idx)


def _route(i01, s01):
    ef = i01.reshape(NA)
    sf = s01.reshape(NA)
    order = jnp.argsort(ef)
    eo_sorted = ef[order]
    counts = jnp.bincount(ef, length=NUM_EXPERT)
    starts = jnp.concatenate([jnp.zeros((1,), counts.dtype), jnp.cumsum(counts)[:-1]])
    cpad = ((counts + TILE - 1) // TILE) * TILE
    pad_off = jnp.concatenate([jnp.zeros((1,), cpad.dtype), jnp.cumsum(cpad)[:-1]])
    j = jnp.arange(NA)
    pj = (pad_off[eo_sorted] + (j - starts[eo_sorted])).astype(jnp.int32)
    disp = jnp.zeros((MPAD,), jnp.int32).at[pj].set((order // TOP_K).astype(jnp.int32))
    wt = jnp.zeros((MPAD,), jnp.float32).at[pj].set(sf[order])
    pp = jnp.zeros((NA,), jnp.int32).at[order].set(pj)
    tile_start = jnp.arange(NTILES) * TILE
    eid = jnp.clip(
        jnp.searchsorted(pad_off, tile_start, side="right") - 1,
        0, NUM_EXPERT - 1).astype(jnp.int32)
    return disp, wt, pp, eid


def kernel(x, params):
    p = params
    patches = x.reshape(BATCH, 3, GRID, PATCH, GRID, PATCH)
    patches = patches.transpose(0, 2, 4, 1, 3, 5).reshape(BATCH * NUM_PATCHES, 3 * PATCH * PATCH)
    emb = _embed(patches, p["patch_w"], p["patch_b"].reshape(1, EMBED))
    emb = emb.reshape(BATCH, NUM_PATCHES, EMBED)
    cls = jnp.broadcast_to(p["cls"], (BATCH, 1, EMBED))
    h = jnp.concatenate([cls, emb], axis=1) + p["pos"]
    h = jnp.pad(h, ((0, 0), (0, TP - T), (0, 0))).reshape(N, EMBED)

    for blk in p["blocks"]:
        r1 = lambda a: a.reshape(1, -1)
        qkv = _ln_qkv(h, r1(blk["ln1_w"]), r1(blk["ln1_b"]),
                      blk["qkv_w"], r1(blk["qkv_b"]))
        ao = _attn(qkv)
        if "w1" in blk:
            h2, xn, i01, s01 = _proj_gate(
                ao, h, blk["proj_w"], r1(blk["proj_b"]),
                r1(blk["ln2_w"]), r1(blk["ln2_b"]),
                blk["gate_w"], r1(blk["gate_b"]))
            disp, wt, pp, eid = _route(i01, s01)
            x_sorted = _gather_rows(xn, disp)
            eo = _ggemm(x_sorted, blk["w1"], blk["b1"].reshape(NUM_EXPERT, 1, HID),
                        blk["w2"], blk["b2"].reshape(NUM_EXPERT, 1, EMBED),
                        wt.reshape(MPAD, 1), eid)
            g = _gather_rows(eo, pp)
            h = _combine(h2, g)
        else:
            h = _proj_mlp(ao, h, blk["proj_w"], r1(blk["proj_b"]),
                          r1(blk["ln2_w"]), r1(blk["ln2_b"]),
                          blk["fc1_w"], r1(blk["fc1_b"]),
                          blk["fc2_w"], r1(blk["fc2_b"]))

    hcls = h.reshape(BATCH, TP, EMBED)[:, 0, :]
    return _head(hcls, r1(p["norm_w"]), r1(p["norm_b"]),
                 p["head_w"], r1(p["head_b"]))


# speedup vs baseline: 1.9286x; 1.0140x over previous
import functools

import jax
import jax.numpy as jnp
from jax import lax
from jax.experimental import pallas as pl
from jax.experimental.pallas import tpu as pltpu
from jax.experimental.pallas import tpu_sc as plsc

EMBED = 384
HEADS = 12
DH = EMBED // HEADS
HID = 1536
NUM_EXPERT = 8
TOP_K = 2
PATCH = 16
GRID = 14
NUM_PATCHES = GRID * GRID
NUM_CLASSES = 1000
BATCH = 16
T = NUM_PATCHES + 1
TP = 200
N = BATCH * TP
TM = 320
NA = N * TOP_K
TILE = 256
NTILES = NA // TILE + NUM_EXPERT
MPAD = NTILES * TILE
NEG = -1e30


def _mm(a, b):
    return lax.dot_general(
        a.astype(jnp.bfloat16), b.astype(jnp.bfloat16),
        (((a.ndim - 1,), (0,)), ((), ())),
        preferred_element_type=jnp.float32)


def _mm_hi(a, b):
    return lax.dot_general(
        a, b, (((a.ndim - 1,), (0,)), ((), ())),
        precision=lax.Precision.HIGHEST, preferred_element_type=jnp.float32)


def _ln(x, w, b):
    mu = jnp.mean(x, axis=-1, keepdims=True)
    xc = x - mu
    var = jnp.mean(xc * xc, axis=-1, keepdims=True)
    return xc * lax.rsqrt(var + 1e-5) * w + b


def _gelu(x):
    return 0.5 * x * (1.0 + lax.erf(x * 0.7071067811865476))


def _embed_body(p_ref, w_ref, b_ref, o_ref):
    o_ref[...] = _mm(p_ref[...], w_ref[...]) + b_ref[...]


def _embed(patches, w, b):
    M = patches.shape[0]
    tm = 392
    return pl.pallas_call(
        _embed_body,
        grid=(M // tm,),
        in_specs=[
            pl.BlockSpec((tm, 3 * PATCH * PATCH), lambda i: (i, 0)),
            pl.BlockSpec((3 * PATCH * PATCH, EMBED), lambda i: (0, 0)),
            pl.BlockSpec((1, EMBED), lambda i: (0, 0)),
        ],
        out_specs=pl.BlockSpec((tm, EMBED), lambda i: (i, 0)),
        out_shape=jax.ShapeDtypeStruct((M, EMBED), jnp.float32),
    )(patches, w, b)


def _lnmm_body(x_ref, lw_ref, lb_ref, w_ref, b_ref, o_ref):
    xn = _ln(x_ref[...], lw_ref[...], lb_ref[...])
    o_ref[...] = _mm(xn, w_ref[...]) + b_ref[...]


def _ln_qkv(h, lw, lb, w, b):
    return pl.pallas_call(
        _lnmm_body,
        grid=(N // TM,),
        in_specs=[
            pl.BlockSpec((TM, EMBED), lambda i: (i, 0)),
            pl.BlockSpec((1, EMBED), lambda i: (0, 0)),
            pl.BlockSpec((1, EMBED), lambda i: (0, 0)),
            pl.BlockSpec((EMBED, 3 * EMBED), lambda i: (0, 0)),
            pl.BlockSpec((1, 3 * EMBED), lambda i: (0, 0)),
        ],
        out_specs=pl.BlockSpec((TM, 3 * EMBED), lambda i: (i, 0)),
        out_shape=jax.ShapeDtypeStruct((N, 3 * EMBED), jnp.float32),
    )(h, lw, lb, w, b)


def _attn_body(qkv_ref, o_ref):
    scale = DH ** -0.5
    qkv = qkv_ref[0]
    col = lax.broadcasted_iota(jnp.int32, (TP, TP), 1)
    mask = jnp.where(col >= T, NEG, 0.0)
    outs = []
    for h in range(HEADS):
        q = qkv[:, DH * h:DH * (h + 1)]
        k = qkv[:, EMBED + DH * h:EMBED + DH * (h + 1)]
        v = qkv[:, 2 * EMBED + DH * h:2 * EMBED + DH * (h + 1)]
        s = lax.dot_general(
            q.astype(jnp.bfloat16), k.astype(jnp.bfloat16),
            (((1,), (1,)), ((), ())),
            preferred_element_type=jnp.float32) * scale + mask
        s = s - jnp.max(s, axis=-1, keepdims=True)
        p = jnp.exp(s)
        p = p / jnp.sum(p, axis=-1, keepdims=True)
        outs.append(_mm(p, v))
    o_ref[0] = jnp.concatenate(outs, axis=-1)


def _attn(qkv):
    qkv3 = qkv.reshape(BATCH, TP, 3 * EMBED)
    out = pl.pallas_call(
        _attn_body,
        grid=(BATCH,),
        in_specs=[pl.BlockSpec((1, TP, 3 * EMBED), lambda i: (i, 0, 0))],
        out_specs=pl.BlockSpec((1, TP, EMBED), lambda i: (i, 0, 0)),
        out_shape=jax.ShapeDtypeStruct((BATCH, TP, EMBED), jnp.float32),
    )(qkv3)
    return out.reshape(N, EMBED)


def _proj_mlp_body(ao_ref, h_ref, pw_ref, pb_ref, lw_ref, lb_ref,
                   w1_ref, b1_ref, w2_ref, b2_ref, o_ref):
    h2 = h_ref[...] + _mm(ao_ref[...], pw_ref[...]) + pb_ref[...]
    xn = _ln(h2, lw_ref[...], lb_ref[...])
    hmid = _gelu(_mm(xn, w1_ref[...]) + b1_ref[...])
    o_ref[...] = h2 + _mm(hmid, w2_ref[...]) + b2_ref[...]


def _proj_mlp(ao, h, pw, pb, lw, lb, w1, b1, w2, b2):
    return pl.pallas_call(
        _proj_mlp_body,
        grid=(N // TM,),
        in_specs=[
            pl.BlockSpec((TM, EMBED), lambda i: (i, 0)),
            pl.BlockSpec((TM, EMBED), lambda i: (i, 0)),
            pl.BlockSpec((EMBED, EMBED), lambda i: (0, 0)),
            pl.BlockSpec((1, EMBED), lambda i: (0, 0)),
            pl.BlockSpec((1, EMBED), lambda i: (0, 0)),
            pl.BlockSpec((1, EMBED), lambda i: (0, 0)),
            pl.BlockSpec((EMBED, HID), lambda i: (0, 0)),
            pl.BlockSpec((1, HID), lambda i: (0, 0)),
            pl.BlockSpec((HID, EMBED), lambda i: (0, 0)),
            pl.BlockSpec((1, EMBED), lambda i: (0, 0)),
        ],
        out_specs=pl.BlockSpec((TM, EMBED), lambda i: (i, 0)),
        out_shape=jax.ShapeDtypeStruct((N, EMBED), jnp.float32),
    )(ao, h, pw, pb, lw, lb, w1, b1, w2, b2)


def _proj_gate_body(ao_ref, h_ref, pw_ref, pb_ref, lw_ref, lb_ref,
                    gw_ref, gb_ref, h2_ref, xn_ref, i01_ref, s01_ref):
    h2 = h_ref[...] + _mm(ao_ref[...], pw_ref[...]) + pb_ref[...]
    xn = _ln(h2, lw_ref[...], lb_ref[...])
    h2_ref[...] = h2
    xn_ref[...] = xn
    logits = _mm_hi(xn, gw_ref[...]) + gb_ref[...]
    iot = lax.broadcasted_iota(jnp.int32, logits.shape, 1)
    m0 = jnp.max(logits, axis=-1, keepdims=True)
    i0 = jnp.min(jnp.where(logits >= m0, iot, NUM_EXPERT), axis=-1, keepdims=True)
    l1 = jnp.where(iot == i0, NEG, logits)
    m1 = jnp.max(l1, axis=-1, keepdims=True)
    i1 = jnp.min(jnp.where(l1 >= m1, iot, NUM_EXPERT), axis=-1, keepdims=True)
    e1 = jnp.exp(m1 - m0)
    s0 = 1.0 / (1.0 + e1)
    i01_ref[...] = jnp.concatenate([i0, i1], axis=-1)
    s01_ref[...] = jnp.concatenate([s0, 1.0 - s0], axis=-1)


def _proj_gate(ao, h, pw, pb, lw, lb, gw, gb):
    return pl.pallas_call(
        _proj_gate_body,
        grid=(N // TM,),
        in_specs=[
            pl.BlockSpec((TM, EMBED), lambda i: (i, 0)),
            pl.BlockSpec((TM, EMBED), lambda i: (i, 0)),
            pl.BlockSpec((EMBED, EMBED), lambda i: (0, 0)),
            pl.BlockSpec((1, EMBED), lambda i: (0, 0)),
            pl.BlockSpec((1, EMBED), lambda i: (0, 0)),
            pl.BlockSpec((1, EMBED), lambda i: (0, 0)),
            pl.BlockSpec((EMBED, NUM_EXPERT), lambda i: (0, 0)),
            pl.BlockSpec((1, NUM_EXPERT), lambda i: (0, 0)),
        ],
        out_specs=[
            pl.BlockSpec((TM, EMBED), lambda i: (i, 0)),
            pl.BlockSpec((TM, EMBED), lambda i: (i, 0)),
            pl.BlockSpec((TM, TOP_K), lambda i: (i, 0)),
            pl.BlockSpec((TM, TOP_K), lambda i: (i, 0)),
        ],
        out_shape=[
            jax.ShapeDtypeStruct((N, EMBED), jnp.float32),
            jax.ShapeDtypeStruct((N, EMBED), jnp.float32),
            jax.ShapeDtypeStruct((N, TOP_K), jnp.int32),
            jax.ShapeDtypeStruct((N, TOP_K), jnp.float32),
        ],
    )(ao, h, pw, pb, lw, lb, gw, gb)


def _ggemm_body(eid_ref, x_ref, w1_ref, b1_ref, w2_ref, b2_ref, wt_ref, o_ref):
    x = x_ref[...]
    hmid = _gelu(_mm(x, w1_ref[0]) + b1_ref[0])
    o = _mm(hmid, w2_ref[0]) + b2_ref[0]
    o_ref[...] = o * wt_ref[...]


def _ggemm(x_sorted, w1, b1, w2, b2, wt, eid):
    grid_spec = pltpu.PrefetchScalarGridSpec(
        num_scalar_prefetch=1,
        grid=(NTILES,),
        in_specs=[
            pl.BlockSpec((TILE, EMBED), lambda g, eid: (g, 0)),
            pl.BlockSpec((1, EMBED, HID), lambda g, eid: (eid[g], 0, 0)),
            pl.BlockSpec((1, 1, HID), lambda g, eid: (eid[g], 0, 0)),
            pl.BlockSpec((1, HID, EMBED), lambda g, eid: (eid[g], 0, 0)),
            pl.BlockSpec((1, 1, EMBED), lambda g, eid: (eid[g], 0, 0)),
            pl.BlockSpec((TILE, 1), lambda g, eid: (g, 0)),
        ],
        out_specs=pl.BlockSpec((TILE, EMBED), lambda g, eid: (g, 0)),
    )
    return pl.pallas_call(
        _ggemm_body,
        grid_spec=grid_spec,
        out_shape=jax.ShapeDtypeStruct((MPAD, EMBED), jnp.float32),
    )(eid, x_sorted, w1, b1, w2, b2, wt)


def _combine_body(h2_ref, g_ref, o_ref):
    o_ref[...] = h2_ref[...] + g_ref[:, 0, :] + g_ref[:, 1, :]


def _combine(h2, g):
    g3 = g.reshape(N, TOP_K, EMBED)
    return pl.pallas_call(
        _combine_body,
        grid=(N // TM,),
        in_specs=[
            pl.BlockSpec((TM, EMBED), lambda i: (i, 0)),
            pl.BlockSpec((TM, TOP_K, EMBED), lambda i: (i, 0, 0)),
        ],
        out_specs=pl.BlockSpec((TM, EMBED), lambda i: (i, 0)),
        out_shape=jax.ShapeDtypeStruct((N, EMBED), jnp.float32),
    )(h2, g3)


def _head_body(x_ref, lw_ref, lb_ref, w_ref, b_ref, o_ref):
    xn = _ln(x_ref[...], lw_ref[...], lb_ref[...])
    o_ref[...] = _mm(xn, w_ref[...]) + b_ref[...]


def _head(hcls, lw, lb, w, b):
    return pl.pallas_call(
        _head_body,
        in_specs=[
            pl.BlockSpec((BATCH, EMBED), lambda: (0, 0)),
            pl.BlockSpec((1, EMBED), lambda: (0, 0)),
            pl.BlockSpec((1, EMBED), lambda: (0, 0)),
            pl.BlockSpec((EMBED, NUM_CLASSES), lambda: (0, 0)),
            pl.BlockSpec((1, NUM_CLASSES), lambda: (0, 0)),
        ],
        out_specs=pl.BlockSpec((BATCH, NUM_CLASSES), lambda: (0, 0)),
        out_shape=jax.ShapeDtypeStruct((BATCH, NUM_CLASSES), jnp.float32),
    )(hcls, lw, lb, w, b)


@functools.cache
def _sc_gather_fn(rows_out, table_rows):
    NW = 32
    per_w = rows_out // NW
    c0 = 8
    for c in range(8, 129, 8):
        if per_w % c == 0:
            c0 = c
    nch = per_w // c0
    mesh = plsc.VectorSubcoreMesh(core_axis_name="c", subcore_axis_name="s")

    @functools.partial(
        pl.kernel, mesh=mesh,
        out_type=jax.ShapeDtypeStruct((rows_out, EMBED), jnp.float32),
        scratch_types=[
            pltpu.VMEM((per_w,), jnp.int32),
            pltpu.VMEM((per_w, EMBED), jnp.float32),
            pltpu.SemaphoreType.DMA,
        ],
    )
    def k(table_hbm, idx_hbm, out_hbm, idx_v, rows_v, sem):
        wid = lax.axis_index("s") * 2 + lax.axis_index("c")
        base = wid * per_w
        pltpu.sync_copy(idx_hbm.at[pl.ds(base, per_w)], idx_v)
        cps = [
            pltpu.async_copy(
                table_hbm.at[idx_v.at[pl.ds(c * c0, c0)]],
                rows_v.at[pl.ds(c * c0, c0)], sem)
            for c in range(nch)
        ]
        for cp in cps:
            cp.wait()
        pltpu.sync_copy(rows_v, out_hbm.at[pl.ds(base, per_w)])

    return k


def _gather_rows(table, idx):
    return _sc_gather_fn(idx.shape[0], table.shape[0])(table, idx)


_RTILE = 256


def _rank_body(e_ref, r_ref, cnt_ref, carry):
    g = pl.program_id(0)

    @pl.when(g == 0)
    def _():
        carry[...] = jnp.zeros_like(carry)

    e = e_ref[...]
    iot = lax.broadcasted_iota(jnp.int32, (_RTILE, NUM_EXPERT), 1)
    oh = (e == iot).astype(jnp.float32)
    row = lax.broadcasted_iota(jnp.int32, (_RTILE, _RTILE), 0)
    col = lax.broadcasted_iota(jnp.int32, (_RTILE, _RTILE), 1)
    tri = (row > col).astype(jnp.float32)
    within = _mm_hi(tri, oh)
    r = jnp.sum(oh * (carry[...] + within), axis=1, keepdims=True)
    r_ref[...] = r
    carry[...] = carry[...] + jnp.sum(oh, axis=0, keepdims=True)
    cnt_ref[...] = carry[...]


def _rank_scan(ef):
    r, cnt = pl.pallas_call(
        _rank_body,
        grid=(NA // _RTILE,),
        in_specs=[pl.BlockSpec((_RTILE, 1), lambda i: (i, 0))],
        out_specs=[
            pl.BlockSpec((_RTILE, 1), lambda i: (i, 0)),
            pl.BlockSpec((1, NUM_EXPERT), lambda i: (0, 0)),
        ],
        out_shape=[
            jax.ShapeDtypeStruct((NA, 1), jnp.float32),
            jax.ShapeDtypeStruct((1, NUM_EXPERT), jnp.float32),
        ],
        scratch_shapes=[pltpu.VMEM((1, NUM_EXPERT), jnp.float32)],
    )(ef.reshape(NA, 1))
    return r.reshape(NA).astype(jnp.int32), cnt.reshape(NUM_EXPERT).astype(jnp.int32)


def _route(i01, s01):
    ef = i01.reshape(NA)
    sf = s01.reshape(NA)
    r, counts = _rank_scan(ef)
    cpad = ((counts + TILE - 1) // TILE) * TILE
    pad_off = jnp.concatenate([jnp.zeros((1,), cpad.dtype), jnp.cumsum(cpad)[:-1]])
    pp = (pad_off[ef] + r).astype(jnp.int32)
    tok = (jnp.arange(NA, dtype=jnp.int32) // TOP_K)
    disp = jnp.zeros((MPAD,), jnp.int32).at[pp].set(tok)
    wt = jnp.zeros((MPAD,), jnp.float32).at[pp].set(sf)
    tile_start = jnp.arange(NTILES) * TILE
    eid = jnp.clip(
        jnp.searchsorted(pad_off, tile_start, side="right") - 1,
        0, NUM_EXPERT - 1).astype(jnp.int32)
    return disp, wt, pp, eid


def kernel(x, params):
    p = params
    patches = x.reshape(BATCH, 3, GRID, PATCH, GRID, PATCH)
    patches = patches.transpose(0, 2, 4, 1, 3, 5).reshape(BATCH * NUM_PATCHES, 3 * PATCH * PATCH)
    emb = _embed(patches, p["patch_w"], p["patch_b"].reshape(1, EMBED))
    emb = emb.reshape(BATCH, NUM_PATCHES, EMBED)
    cls = jnp.broadcast_to(p["cls"], (BATCH, 1, EMBED))
    h = jnp.concatenate([cls, emb], axis=1) + p["pos"]
    h = jnp.pad(h, ((0, 0), (0, TP - T), (0, 0))).reshape(N, EMBED)

    for blk in p["blocks"]:
        r1 = lambda a: a.reshape(1, -1)
        qkv = _ln_qkv(h, r1(blk["ln1_w"]), r1(blk["ln1_b"]),
                      blk["qkv_w"], r1(blk["qkv_b"]))
        ao = _attn(qkv)
        if "w1" in blk:
            h2, xn, i01, s01 = _proj_gate(
                ao, h, blk["proj_w"], r1(blk["proj_b"]),
                r1(blk["ln2_w"]), r1(blk["ln2_b"]),
                blk["gate_w"], r1(blk["gate_b"]))
            disp, wt, pp, eid = _route(i01, s01)
            x_sorted = _gather_rows(xn, disp)
            eo = _ggemm(x_sorted, blk["w1"], blk["b1"].reshape(NUM_EXPERT, 1, HID),
                        blk["w2"], blk["b2"].reshape(NUM_EXPERT, 1, EMBED),
                        wt.reshape(MPAD, 1), eid)
            g = _gather_rows(eo, pp)
            h = _combine(h2, g)
        else:
            h = _proj_mlp(ao, h, blk["proj_w"], r1(blk["proj_b"]),
                          r1(blk["ln2_w"]), r1(blk["ln2_b"]),
                          blk["fc1_w"], r1(blk["fc1_b"]),
                          blk["fc2_w"], r1(blk["fc2_b"]))

    hcls = h.reshape(BATCH, TP, EMBED)[:, 0, :]
    return _head(hcls, r1(p["norm_w"]), r1(p["norm_b"]),
                 p["head_w"], r1(p["head_b"]))


# speedup vs baseline: 2.4102x; 1.2497x over previous
import functools

import jax
import jax.numpy as jnp
from jax import lax
from jax.experimental import pallas as pl
from jax.experimental.pallas import tpu as pltpu
from jax.experimental.pallas import tpu_sc as plsc

EMBED = 384
HEADS = 12
DH = EMBED // HEADS
HID = 1536
NUM_EXPERT = 8
TOP_K = 2
PATCH = 16
GRID = 14
NUM_PATCHES = GRID * GRID
NUM_CLASSES = 1000
BATCH = 16
T = NUM_PATCHES + 1
TP = 200
N = BATCH * TP
TM = 320
NA = N * TOP_K
TILE = 256
NTILES = NA // TILE + NUM_EXPERT
MPAD = NTILES * TILE
NEG = -1e30


def _mm(a, b):
    return lax.dot_general(
        a.astype(jnp.bfloat16), b.astype(jnp.bfloat16),
        (((a.ndim - 1,), (0,)), ((), ())),
        preferred_element_type=jnp.float32)


def _mm_hi(a, b):
    return lax.dot_general(
        a, b, (((a.ndim - 1,), (0,)), ((), ())),
        precision=lax.Precision.HIGHEST, preferred_element_type=jnp.float32)


def _ln(x, w, b):
    mu = jnp.mean(x, axis=-1, keepdims=True)
    xc = x - mu
    var = jnp.mean(xc * xc, axis=-1, keepdims=True)
    return xc * lax.rsqrt(var + 1e-5) * w + b


def _gelu(x):
    return 0.5 * x * (1.0 + lax.erf(x * 0.7071067811865476))


def _embed_body(p_ref, w_ref, b_ref, o_ref):
    o_ref[...] = _mm(p_ref[...], w_ref[...]) + b_ref[...]


def _embed(patches, w, b):
    M = patches.shape[0]
    tm = 392
    return pl.pallas_call(
        _embed_body,
        grid=(M // tm,),
        in_specs=[
            pl.BlockSpec((tm, 3 * PATCH * PATCH), lambda i: (i, 0)),
            pl.BlockSpec((3 * PATCH * PATCH, EMBED), lambda i: (0, 0)),
            pl.BlockSpec((1, EMBED), lambda i: (0, 0)),
        ],
        out_specs=pl.BlockSpec((tm, EMBED), lambda i: (i, 0)),
        out_shape=jax.ShapeDtypeStruct((M, EMBED), jnp.float32),
    )(patches, w, b)


def _lnmm_body(x_ref, lw_ref, lb_ref, w_ref, b_ref, o_ref):
    xn = _ln(x_ref[...], lw_ref[...], lb_ref[...])
    o_ref[...] = _mm(xn, w_ref[...]) + b_ref[...]


def _ln_qkv(h, lw, lb, w, b):
    return pl.pallas_call(
        _lnmm_body,
        grid=(N // TM,),
        in_specs=[
            pl.BlockSpec((TM, EMBED), lambda i: (i, 0)),
            pl.BlockSpec((1, EMBED), lambda i: (0, 0)),
            pl.BlockSpec((1, EMBED), lambda i: (0, 0)),
            pl.BlockSpec((EMBED, 3 * EMBED), lambda i: (0, 0)),
            pl.BlockSpec((1, 3 * EMBED), lambda i: (0, 0)),
        ],
        out_specs=pl.BlockSpec((TM, 3 * EMBED), lambda i: (i, 0)),
        out_shape=jax.ShapeDtypeStruct((N, 3 * EMBED), jnp.float32),
    )(h, lw, lb, w, b)


def _attn_body(qkv_ref, o_ref):
    scale = DH ** -0.5
    qkv = qkv_ref[0]
    col = lax.broadcasted_iota(jnp.int32, (TP, TP), 1)
    mask = jnp.where(col >= T, NEG, 0.0)
    outs = []
    for h in range(HEADS):
        q = qkv[:, DH * h:DH * (h + 1)]
        k = qkv[:, EMBED + DH * h:EMBED + DH * (h + 1)]
        v = qkv[:, 2 * EMBED + DH * h:2 * EMBED + DH * (h + 1)]
        s = lax.dot_general(
            q.astype(jnp.bfloat16), k.astype(jnp.bfloat16),
            (((1,), (1,)), ((), ())),
            preferred_element_type=jnp.float32) * scale + mask
        s = s - jnp.max(s, axis=-1, keepdims=True)
        p = jnp.exp(s)
        p = p / jnp.sum(p, axis=-1, keepdims=True)
        outs.append(_mm(p, v))
    o_ref[0] = jnp.concatenate(outs, axis=-1)


def _attn(qkv):
    qkv3 = qkv.reshape(BATCH, TP, 3 * EMBED)
    out = pl.pallas_call(
        _attn_body,
        grid=(BATCH,),
        in_specs=[pl.BlockSpec((1, TP, 3 * EMBED), lambda i: (i, 0, 0))],
        out_specs=pl.BlockSpec((1, TP, EMBED), lambda i: (i, 0, 0)),
        out_shape=jax.ShapeDtypeStruct((BATCH, TP, EMBED), jnp.float32),
    )(qkv3)
    return out.reshape(N, EMBED)


def _proj_mlp_body(ao_ref, h_ref, pw_ref, pb_ref, lw_ref, lb_ref,
                   w1_ref, b1_ref, w2_ref, b2_ref, o_ref):
    h2 = h_ref[...] + _mm(ao_ref[...], pw_ref[...]) + pb_ref[...]
    xn = _ln(h2, lw_ref[...], lb_ref[...])
    hmid = _gelu(_mm(xn, w1_ref[...]) + b1_ref[...])
    o_ref[...] = h2 + _mm(hmid, w2_ref[...]) + b2_ref[...]


def _proj_mlp(ao, h, pw, pb, lw, lb, w1, b1, w2, b2):
    return pl.pallas_call(
        _proj_mlp_body,
        grid=(N // TM,),
        in_specs=[
            pl.BlockSpec((TM, EMBED), lambda i: (i, 0)),
            pl.BlockSpec((TM, EMBED), lambda i: (i, 0)),
            pl.BlockSpec((EMBED, EMBED), lambda i: (0, 0)),
            pl.BlockSpec((1, EMBED), lambda i: (0, 0)),
            pl.BlockSpec((1, EMBED), lambda i: (0, 0)),
            pl.BlockSpec((1, EMBED), lambda i: (0, 0)),
            pl.BlockSpec((EMBED, HID), lambda i: (0, 0)),
            pl.BlockSpec((1, HID), lambda i: (0, 0)),
            pl.BlockSpec((HID, EMBED), lambda i: (0, 0)),
            pl.BlockSpec((1, EMBED), lambda i: (0, 0)),
        ],
        out_specs=pl.BlockSpec((TM, EMBED), lambda i: (i, 0)),
        out_shape=jax.ShapeDtypeStruct((N, EMBED), jnp.float32),
    )(ao, h, pw, pb, lw, lb, w1, b1, w2, b2)


def _proj_gate_body(ao_ref, h_ref, pw_ref, pb_ref, lw_ref, lb_ref,
                    gw_ref, gb_ref, h2_ref, xn_ref, i01_ref, s01_ref):
    h2 = h_ref[...] + _mm(ao_ref[...], pw_ref[...]) + pb_ref[...]
    xn = _ln(h2, lw_ref[...], lb_ref[...])
    h2_ref[...] = h2
    xn_ref[...] = xn
    logits = _mm_hi(xn, gw_ref[...]) + gb_ref[...]
    iot = lax.broadcasted_iota(jnp.int32, logits.shape, 1)
    m0 = jnp.max(logits, axis=-1, keepdims=True)
    i0 = jnp.min(jnp.where(logits >= m0, iot, NUM_EXPERT), axis=-1, keepdims=True)
    l1 = jnp.where(iot == i0, NEG, logits)
    m1 = jnp.max(l1, axis=-1, keepdims=True)
    i1 = jnp.min(jnp.where(l1 >= m1, iot, NUM_EXPERT), axis=-1, keepdims=True)
    e1 = jnp.exp(m1 - m0)
    s0 = 1.0 / (1.0 + e1)
    i01_ref[...] = jnp.concatenate([i0, i1], axis=-1)
    s01_ref[...] = jnp.concatenate([s0, 1.0 - s0], axis=-1)


def _proj_gate(ao, h, pw, pb, lw, lb, gw, gb):
    return pl.pallas_call(
        _proj_gate_body,
        grid=(N // TM,),
        in_specs=[
            pl.BlockSpec((TM, EMBED), lambda i: (i, 0)),
            pl.BlockSpec((TM, EMBED), lambda i: (i, 0)),
            pl.BlockSpec((EMBED, EMBED), lambda i: (0, 0)),
            pl.BlockSpec((1, EMBED), lambda i: (0, 0)),
            pl.BlockSpec((1, EMBED), lambda i: (0, 0)),
            pl.BlockSpec((1, EMBED), lambda i: (0, 0)),
            pl.BlockSpec((EMBED, NUM_EXPERT), lambda i: (0, 0)),
            pl.BlockSpec((1, NUM_EXPERT), lambda i: (0, 0)),
        ],
        out_specs=[
            pl.BlockSpec((TM, EMBED), lambda i: (i, 0)),
            pl.BlockSpec((TM, EMBED), lambda i: (i, 0)),
            pl.BlockSpec((TM, TOP_K), lambda i: (i, 0)),
            pl.BlockSpec((TM, TOP_K), lambda i: (i, 0)),
        ],
        out_shape=[
            jax.ShapeDtypeStruct((N, EMBED), jnp.float32),
            jax.ShapeDtypeStruct((N, EMBED), jnp.float32),
            jax.ShapeDtypeStruct((N, TOP_K), jnp.int32),
            jax.ShapeDtypeStruct((N, TOP_K), jnp.float32),
        ],
    )(ao, h, pw, pb, lw, lb, gw, gb)


def _ggemm_body(eid_ref, x_ref, w1_ref, b1_ref, w2_ref, b2_ref, o_ref):
    x = x_ref[...]
    hmid = _gelu(_mm(x, w1_ref[0]) + b1_ref[0])
    o_ref[...] = _mm(hmid, w2_ref[0]) + b2_ref[0]


def _ggemm(x_sorted, w1, b1, w2, b2, eid):
    grid_spec = pltpu.PrefetchScalarGridSpec(
        num_scalar_prefetch=1,
        grid=(NTILES,),
        in_specs=[
            pl.BlockSpec((TILE, EMBED), lambda g, eid: (g, 0)),
            pl.BlockSpec((1, EMBED, HID), lambda g, eid: (eid[g], 0, 0)),
            pl.BlockSpec((1, 1, HID), lambda g, eid: (eid[g], 0, 0)),
            pl.BlockSpec((1, HID, EMBED), lambda g, eid: (eid[g], 0, 0)),
            pl.BlockSpec((1, 1, EMBED), lambda g, eid: (eid[g], 0, 0)),
        ],
        out_specs=pl.BlockSpec((TILE, EMBED), lambda g, eid: (g, 0)),
    )
    return pl.pallas_call(
        _ggemm_body,
        grid_spec=grid_spec,
        out_shape=jax.ShapeDtypeStruct((MPAD, EMBED), jnp.float32),
    )(eid, x_sorted, w1, b1, w2, b2)


def _combine_body(h2_ref, g0_ref, g1_ref, s_ref, o_ref):
    s = s_ref[...]
    o_ref[...] = (h2_ref[...] + g0_ref[...] * s[:, 0:1] + g1_ref[...] * s[:, 1:2])


def _combine(h2, g, s01):
    return pl.pallas_call(
        _combine_body,
        grid=(N // TM,),
        in_specs=[
            pl.BlockSpec((TM, EMBED), lambda i: (i, 0)),
            pl.BlockSpec((TM, EMBED), lambda i: (i, 0)),
            pl.BlockSpec((TM, EMBED), lambda i: (i + N // TM, 0)),
            pl.BlockSpec((TM, TOP_K), lambda i: (i, 0)),
        ],
        out_specs=pl.BlockSpec((TM, EMBED), lambda i: (i, 0)),
        out_shape=jax.ShapeDtypeStruct((N, EMBED), jnp.float32),
    )(h2, g, g, s01)


def _head_body(x_ref, lw_ref, lb_ref, w_ref, b_ref, o_ref):
    xn = _ln(x_ref[...], lw_ref[...], lb_ref[...])
    o_ref[...] = _mm(xn, w_ref[...]) + b_ref[...]


def _head(hcls, lw, lb, w, b):
    return pl.pallas_call(
        _head_body,
        in_specs=[
            pl.BlockSpec((BATCH, EMBED), lambda: (0, 0)),
            pl.BlockSpec((1, EMBED), lambda: (0, 0)),
            pl.BlockSpec((1, EMBED), lambda: (0, 0)),
            pl.BlockSpec((EMBED, NUM_CLASSES), lambda: (0, 0)),
            pl.BlockSpec((1, NUM_CLASSES), lambda: (0, 0)),
        ],
        out_specs=pl.BlockSpec((BATCH, NUM_CLASSES), lambda: (0, 0)),
        out_shape=jax.ShapeDtypeStruct((BATCH, NUM_CLASSES), jnp.float32),
    )(hcls, lw, lb, w, b)


_CH = 128
_NCHUNK = NA // _CH
_NW = 32


def _chunk_rows(wid):
    return [wid, wid + _NW]


@functools.cache
def _sc_dispatch_fn():
    mesh = plsc.VectorSubcoreMesh(core_axis_name="c", subcore_axis_name="s")

    @functools.partial(
        pl.kernel, mesh=mesh,
        out_type=jax.ShapeDtypeStruct((MPAD, EMBED), jnp.float32),
        scratch_types=[
            pltpu.VMEM((_CH,), jnp.int32),
            pltpu.VMEM((_CH, EMBED), jnp.float32),
            pltpu.SemaphoreType.DMA,
        ],
    )
    def k(xln_hbm, pp_hbm, xs_hbm, idx_v, rows_v, sem):
        wid = lax.axis_index("s") * 2 + lax.axis_index("c")
        for j in _chunk_rows(wid):
            @pl.when(j < _NCHUNK)
            def _():
                tb = jnp.where(j < _NCHUNK // 2, j * _CH, j * _CH - N)
                pltpu.sync_copy(pp_hbm.at[j], idx_v)
                pltpu.sync_copy(xln_hbm.at[pl.ds(tb, _CH)], rows_v)
                pltpu.async_copy(rows_v, xs_hbm.at[idx_v], sem).wait()

    return k


@functools.cache
def _sc_collect_fn():
    mesh = plsc.VectorSubcoreMesh(core_axis_name="c", subcore_axis_name="s")

    @functools.partial(
        pl.kernel, mesh=mesh,
        out_type=jax.ShapeDtypeStruct((NA, EMBED), jnp.float32),
        scratch_types=[
            pltpu.VMEM((_CH,), jnp.int32),
            pltpu.VMEM((_CH, EMBED), jnp.float32),
            pltpu.SemaphoreType.DMA,
        ],
    )
    def k(eo_hbm, pp_hbm, g_hbm, idx_v, rows_v, sem):
        wid = lax.axis_index("s") * 2 + lax.axis_index("c")
        for j in _chunk_rows(wid):
            @pl.when(j < _NCHUNK)
            def _():
                pltpu.sync_copy(pp_hbm.at[j], idx_v)
                pltpu.async_copy(eo_hbm.at[idx_v], rows_v, sem).wait()
                pltpu.sync_copy(rows_v, g_hbm.at[pl.ds(j * _CH, _CH)])

    return k


def _sc_dispatch(xln, pp2):
    return _sc_dispatch_fn()(xln, pp2)


def _sc_collect(eo, pp2):
    return _sc_collect_fn()(eo, pp2)


_RTILE = 256


def _rank_body(e_ref, r_ref, cnt_ref, carry):
    g = pl.program_id(0)

    @pl.when(g == 0)
    def _():
        carry[...] = jnp.zeros_like(carry)

    e = e_ref[...]
    iot = lax.broadcasted_iota(jnp.int32, (_RTILE, NUM_EXPERT), 1)
    oh = (e == iot).astype(jnp.float32)
    row = lax.broadcasted_iota(jnp.int32, (_RTILE, _RTILE), 0)
    col = lax.broadcasted_iota(jnp.int32, (_RTILE, _RTILE), 1)
    tri = (row > col).astype(jnp.float32)
    within = _mm_hi(tri, oh)
    r = jnp.sum(oh * (carry[...] + within), axis=1, keepdims=True)
    r_ref[...] = r
    carry[...] = carry[...] + jnp.sum(oh, axis=0, keepdims=True)
    cnt_ref[...] = carry[...]


def _rank_scan(ef):
    r, cnt = pl.pallas_call(
        _rank_body,
        grid=(NA // _RTILE,),
        in_specs=[pl.BlockSpec((_RTILE, 1), lambda i: (i, 0))],
        out_specs=[
            pl.BlockSpec((_RTILE, 1), lambda i: (i, 0)),
            pl.BlockSpec((1, NUM_EXPERT), lambda i: (0, 0)),
        ],
        out_shape=[
            jax.ShapeDtypeStruct((NA, 1), jnp.float32),
            jax.ShapeDtypeStruct((1, NUM_EXPERT), jnp.float32),
        ],
        scratch_shapes=[pltpu.VMEM((1, NUM_EXPERT), jnp.float32)],
    )(ef.reshape(NA, 1))
    return r.reshape(NA).astype(jnp.int32), cnt.reshape(NUM_EXPERT).astype(jnp.int32)


def _route(i01):
    ef = jnp.concatenate([i01[:, 0], i01[:, 1]])
    r, counts = _rank_scan(ef)
    cpad = ((counts + TILE - 1) // TILE) * TILE
    pad_off = jnp.concatenate([jnp.zeros((1,), cpad.dtype), jnp.cumsum(cpad)[:-1]])
    pp = (pad_off[ef] + r).astype(jnp.int32)
    tile_start = jnp.arange(NTILES) * TILE
    eid = jnp.clip(
        jnp.searchsorted(pad_off, tile_start, side="right") - 1,
        0, NUM_EXPERT - 1).astype(jnp.int32)
    return pp.reshape(_NCHUNK, _CH), eid


def kernel(x, params):
    p = params
    patches = x.reshape(BATCH, 3, GRID, PATCH, GRID, PATCH)
    patches = patches.transpose(0, 2, 4, 1, 3, 5).reshape(BATCH * NUM_PATCHES, 3 * PATCH * PATCH)
    emb = _embed(patches, p["patch_w"], p["patch_b"].reshape(1, EMBED))
    emb = emb.reshape(BATCH, NUM_PATCHES, EMBED)
    cls = jnp.broadcast_to(p["cls"], (BATCH, 1, EMBED))
    h = jnp.concatenate([cls, emb], axis=1) + p["pos"]
    h = jnp.pad(h, ((0, 0), (0, TP - T), (0, 0))).reshape(N, EMBED)

    for blk in p["blocks"]:
        r1 = lambda a: a.reshape(1, -1)
        qkv = _ln_qkv(h, r1(blk["ln1_w"]), r1(blk["ln1_b"]),
                      blk["qkv_w"], r1(blk["qkv_b"]))
        ao = _attn(qkv)
        if "w1" in blk:
            h2, xn, i01, s01 = _proj_gate(
                ao, h, blk["proj_w"], r1(blk["proj_b"]),
                r1(blk["ln2_w"]), r1(blk["ln2_b"]),
                blk["gate_w"], r1(blk["gate_b"]))
            pp2, eid = _route(i01)
            x_sorted = _sc_dispatch(xn, pp2)
            eo = _ggemm(x_sorted, blk["w1"], blk["b1"].reshape(NUM_EXPERT, 1, HID),
                        blk["w2"], blk["b2"].reshape(NUM_EXPERT, 1, EMBED), eid)
            g = _sc_collect(eo, pp2)
            h = _combine(h2, g, s01)
        else:
            h = _proj_mlp(ao, h, blk["proj_w"], r1(blk["proj_b"]),
                          r1(blk["ln2_w"]), r1(blk["ln2_b"]),
                          blk["fc1_w"], r1(blk["fc1_b"]),
                          blk["fc2_w"], r1(blk["fc2_b"]))

    hcls = h.reshape(BATCH, TP, EMBED)[:, 0, :]
    return _head(hcls, r1(p["norm_w"]), r1(p["norm_b"]),
                 p["head_w"], r1(p["head_b"]))


# speedup vs baseline: 3.0722x; 1.2747x over previous
import functools

import jax
import jax.numpy as jnp
from jax import lax
from jax.experimental import pallas as pl
from jax.experimental.pallas import tpu as pltpu
from jax.experimental.pallas import tpu_sc as plsc

EMBED = 384
HEADS = 12
DH = EMBED // HEADS
HID = 1536
NUM_EXPERT = 8
TOP_K = 2
PATCH = 16
GRID = 14
NUM_PATCHES = GRID * GRID
NUM_CLASSES = 1000
BATCH = 16
T = NUM_PATCHES + 1
TP = 200
N = BATCH * TP
TM = 320
NA = N * TOP_K
TILE = 256
NTILES = NA // TILE + NUM_EXPERT
MPAD = NTILES * TILE
NEG = -1e30


def _mm(a, b):
    return lax.dot_general(
        a.astype(jnp.bfloat16), b.astype(jnp.bfloat16),
        (((a.ndim - 1,), (0,)), ((), ())),
        preferred_element_type=jnp.float32)


def _mm_hi(a, b):
    return lax.dot_general(
        a, b, (((a.ndim - 1,), (0,)), ((), ())),
        precision=lax.Precision.HIGHEST, preferred_element_type=jnp.float32)


def _ln(x, w, b):
    mu = jnp.mean(x, axis=-1, keepdims=True)
    xc = x - mu
    var = jnp.mean(xc * xc, axis=-1, keepdims=True)
    return xc * lax.rsqrt(var + 1e-5) * w + b


def _gelu(x):
    return 0.5 * x * (1.0 + lax.erf(x * 0.7071067811865476))


_GXL = GRID * PATCH
_BGY = BATCH * GRID
_WEXP = GRID * EMBED


def _embed_body(x_ref, w_ref, b_ref, o_ref, wexp, acc):
    c = pl.program_id(0)

    @pl.when(c == 0)
    def _():
        wexp[...] = jnp.zeros_like(wexp)
        acc[...] = jnp.zeros_like(acc)

    xc = x_ref[:, 0]
    for py in range(PATCH):
        wv = w_ref[0, py].astype(jnp.bfloat16)
        for gx in range(GRID):
            wexp[PATCH * gx:PATCH * (gx + 1), EMBED * gx:EMBED * (gx + 1)] = wv
        xs = xc[:, :, py, :].reshape(_BGY, _GXL)
        acc[...] += lax.dot_general(
            xs.astype(jnp.bfloat16), wexp[...],
            (((1,), (0,)), ((), ())), preferred_element_type=jnp.float32)

    @pl.when(c == 2)
    def _():
        o_ref[...] = (acc[...].reshape(_BGY, GRID, EMBED).reshape(
            BATCH * NUM_PATCHES, EMBED) + b_ref[...])


def _embed(x, w, b):
    xr = x.reshape(BATCH, 3, GRID, PATCH, _GXL)
    w4 = w.reshape(3, PATCH, PATCH, EMBED)
    return pl.pallas_call(
        _embed_body,
        grid=(3,),
        in_specs=[
            pl.BlockSpec((BATCH, 1, GRID, PATCH, _GXL), lambda c: (0, c, 0, 0, 0)),
            pl.BlockSpec((1, PATCH, PATCH, EMBED), lambda c: (c, 0, 0, 0)),
            pl.BlockSpec((1, EMBED), lambda c: (0, 0)),
        ],
        out_specs=pl.BlockSpec((BATCH * NUM_PATCHES, EMBED), lambda c: (0, 0)),
        out_shape=jax.ShapeDtypeStruct((BATCH * NUM_PATCHES, EMBED), jnp.float32),
        scratch_shapes=[
            pltpu.VMEM((_GXL, _WEXP), jnp.bfloat16),
            pltpu.VMEM((_BGY, _WEXP), jnp.float32),
        ],
    )(xr, w4, b)


def _lnmm_body(x_ref, lw_ref, lb_ref, w_ref, b_ref, o_ref):
    xn = _ln(x_ref[...], lw_ref[...], lb_ref[...])
    o_ref[...] = _mm(xn, w_ref[...]) + b_ref[...]


def _ln_qkv(h, lw, lb, w, b):
    return pl.pallas_call(
        _lnmm_body,
        grid=(N // TM,),
        in_specs=[
            pl.BlockSpec((TM, EMBED), lambda i: (i, 0)),
            pl.BlockSpec((1, EMBED), lambda i: (0, 0)),
            pl.BlockSpec((1, EMBED), lambda i: (0, 0)),
            pl.BlockSpec((EMBED, 3 * EMBED), lambda i: (0, 0)),
            pl.BlockSpec((1, 3 * EMBED), lambda i: (0, 0)),
        ],
        out_specs=pl.BlockSpec((TM, 3 * EMBED), lambda i: (i, 0)),
        out_shape=jax.ShapeDtypeStruct((N, 3 * EMBED), jnp.float32),
    )(h, lw, lb, w, b)


def _attn_body(qkv_ref, o_ref):
    scale = DH ** -0.5
    qkv = qkv_ref[0]
    col = lax.broadcasted_iota(jnp.int32, (TP, TP), 1)
    mask = jnp.where(col >= T, NEG, 0.0)
    outs = []
    for h in range(HEADS):
        q = qkv[:, DH * h:DH * (h + 1)]
        k = qkv[:, EMBED + DH * h:EMBED + DH * (h + 1)]
        v = qkv[:, 2 * EMBED + DH * h:2 * EMBED + DH * (h + 1)]
        s = lax.dot_general(
            q.astype(jnp.bfloat16), k.astype(jnp.bfloat16),
            (((1,), (1,)), ((), ())),
            preferred_element_type=jnp.float32) * scale + mask
        s = s - jnp.max(s, axis=-1, keepdims=True)
        p = jnp.exp(s)
        p = p / jnp.sum(p, axis=-1, keepdims=True)
        outs.append(_mm(p, v))
    o_ref[0] = jnp.concatenate(outs, axis=-1)


def _attn(qkv):
    qkv3 = qkv.reshape(BATCH, TP, 3 * EMBED)
    out = pl.pallas_call(
        _attn_body,
        grid=(BATCH,),
        in_specs=[pl.BlockSpec((1, TP, 3 * EMBED), lambda i: (i, 0, 0))],
        out_specs=pl.BlockSpec((1, TP, EMBED), lambda i: (i, 0, 0)),
        out_shape=jax.ShapeDtypeStruct((BATCH, TP, EMBED), jnp.float32),
    )(qkv3)
    return out.reshape(N, EMBED)


def _proj_mlp_body(ao_ref, h_ref, pw_ref, pb_ref, lw_ref, lb_ref,
                   w1_ref, b1_ref, w2_ref, b2_ref, o_ref):
    h2 = h_ref[...] + _mm(ao_ref[...], pw_ref[...]) + pb_ref[...]
    xn = _ln(h2, lw_ref[...], lb_ref[...])
    hmid = _gelu(_mm(xn, w1_ref[...]) + b1_ref[...])
    o_ref[...] = h2 + _mm(hmid, w2_ref[...]) + b2_ref[...]


def _proj_mlp(ao, h, pw, pb, lw, lb, w1, b1, w2, b2):
    return pl.pallas_call(
        _proj_mlp_body,
        grid=(N // TM,),
        in_specs=[
            pl.BlockSpec((TM, EMBED), lambda i: (i, 0)),
            pl.BlockSpec((TM, EMBED), lambda i: (i, 0)),
            pl.BlockSpec((EMBED, EMBED), lambda i: (0, 0)),
            pl.BlockSpec((1, EMBED), lambda i: (0, 0)),
            pl.BlockSpec((1, EMBED), lambda i: (0, 0)),
            pl.BlockSpec((1, EMBED), lambda i: (0, 0)),
            pl.BlockSpec((EMBED, HID), lambda i: (0, 0)),
            pl.BlockSpec((1, HID), lambda i: (0, 0)),
            pl.BlockSpec((HID, EMBED), lambda i: (0, 0)),
            pl.BlockSpec((1, EMBED), lambda i: (0, 0)),
        ],
        out_specs=pl.BlockSpec((TM, EMBED), lambda i: (i, 0)),
        out_shape=jax.ShapeDtypeStruct((N, EMBED), jnp.float32),
    )(ao, h, pw, pb, lw, lb, w1, b1, w2, b2)


def _proj_gate_body(ao_ref, h_ref, pw_ref, pb_ref, lw_ref, lb_ref,
                    gw_ref, gb_ref, h2_ref, xn_ref, i01_ref, s01_ref):
    h2 = h_ref[...] + _mm(ao_ref[...], pw_ref[...]) + pb_ref[...]
    xn = _ln(h2, lw_ref[...], lb_ref[...])
    h2_ref[...] = h2
    xn_ref[...] = xn
    logits = _mm_hi(xn, gw_ref[...]) + gb_ref[...]
    iot = lax.broadcasted_iota(jnp.int32, logits.shape, 1)
    m0 = jnp.max(logits, axis=-1, keepdims=True)
    i0 = jnp.min(jnp.where(logits >= m0, iot, NUM_EXPERT), axis=-1, keepdims=True)
    l1 = jnp.where(iot == i0, NEG, logits)
    m1 = jnp.max(l1, axis=-1, keepdims=True)
    i1 = jnp.min(jnp.where(l1 >= m1, iot, NUM_EXPERT), axis=-1, keepdims=True)
    e1 = jnp.exp(m1 - m0)
    s0 = 1.0 / (1.0 + e1)
    i01_ref[...] = jnp.concatenate([i0, i1], axis=-1)
    s01_ref[...] = jnp.concatenate([s0, 1.0 - s0], axis=-1)


def _proj_gate(ao, h, pw, pb, lw, lb, gw, gb):
    return pl.pallas_call(
        _proj_gate_body,
        grid=(N // TM,),
        in_specs=[
            pl.BlockSpec((TM, EMBED), lambda i: (i, 0)),
            pl.BlockSpec((TM, EMBED), lambda i: (i, 0)),
            pl.BlockSpec((EMBED, EMBED), lambda i: (0, 0)),
            pl.BlockSpec((1, EMBED), lambda i: (0, 0)),
            pl.BlockSpec((1, EMBED), lambda i: (0, 0)),
            pl.BlockSpec((1, EMBED), lambda i: (0, 0)),
            pl.BlockSpec((EMBED, NUM_EXPERT), lambda i: (0, 0)),
            pl.BlockSpec((1, NUM_EXPERT), lambda i: (0, 0)),
        ],
        out_specs=[
            pl.BlockSpec((TM, EMBED), lambda i: (i, 0)),
            pl.BlockSpec((TM, EMBED), lambda i: (i, 0)),
            pl.BlockSpec((TM, TOP_K), lambda i: (i, 0)),
            pl.BlockSpec((TM, TOP_K), lambda i: (i, 0)),
        ],
        out_shape=[
            jax.ShapeDtypeStruct((N, EMBED), jnp.float32),
            jax.ShapeDtypeStruct((N, EMBED), jnp.float32),
            jax.ShapeDtypeStruct((N, TOP_K), jnp.int32),
            jax.ShapeDtypeStruct((N, TOP_K), jnp.float32),
        ],
    )(ao, h, pw, pb, lw, lb, gw, gb)


def _ggemm_body(eid_ref, x_ref, w1_ref, b1_ref, w2_ref, b2_ref, o_ref):
    x = x_ref[...]
    hmid = _gelu(_mm(x, w1_ref[0]) + b1_ref[0])
    o_ref[...] = _mm(hmid, w2_ref[0]) + b2_ref[0]


def _ggemm(x_sorted, w1, b1, w2, b2, eid):
    grid_spec = pltpu.PrefetchScalarGridSpec(
        num_scalar_prefetch=1,
        grid=(NTILES,),
        in_specs=[
            pl.BlockSpec((TILE, EMBED), lambda g, eid: (g, 0)),
            pl.BlockSpec((1, EMBED, HID), lambda g, eid: (eid[g], 0, 0)),
            pl.BlockSpec((1, 1, HID), lambda g, eid: (eid[g], 0, 0)),
            pl.BlockSpec((1, HID, EMBED), lambda g, eid: (eid[g], 0, 0)),
            pl.BlockSpec((1, 1, EMBED), lambda g, eid: (eid[g], 0, 0)),
        ],
        out_specs=pl.BlockSpec((TILE, EMBED), lambda g, eid: (g, 0)),
    )
    return pl.pallas_call(
        _ggemm_body,
        grid_spec=grid_spec,
        out_shape=jax.ShapeDtypeStruct((MPAD, EMBED), jnp.float32),
    )(eid, x_sorted, w1, b1, w2, b2)


def _combine_body(h2_ref, g0_ref, g1_ref, s_ref, o_ref):
    s = s_ref[...]
    o_ref[...] = (h2_ref[...] + g0_ref[...] * s[:, 0:1] + g1_ref[...] * s[:, 1:2])


def _combine(h2, g, s01):
    return pl.pallas_call(
        _combine_body,
        grid=(N // TM,),
        in_specs=[
            pl.BlockSpec((TM, EMBED), lambda i: (i, 0)),
            pl.BlockSpec((TM, EMBED), lambda i: (i, 0)),
            pl.BlockSpec((TM, EMBED), lambda i: (i + N // TM, 0)),
            pl.BlockSpec((TM, TOP_K), lambda i: (i, 0)),
        ],
        out_specs=pl.BlockSpec((TM, EMBED), lambda i: (i, 0)),
        out_shape=jax.ShapeDtypeStruct((N, EMBED), jnp.float32),
    )(h2, g, g, s01)


def _head_body(x_ref, lw_ref, lb_ref, w_ref, b_ref, o_ref):
    xn = _ln(x_ref[...], lw_ref[...], lb_ref[...])
    o_ref[...] = _mm(xn, w_ref[...]) + b_ref[...]


def _head(hcls, lw, lb, w, b):
    return pl.pallas_call(
        _head_body,
        in_specs=[
            pl.BlockSpec((BATCH, EMBED), lambda: (0, 0)),
            pl.BlockSpec((1, EMBED), lambda: (0, 0)),
            pl.BlockSpec((1, EMBED), lambda: (0, 0)),
            pl.BlockSpec((EMBED, NUM_CLASSES), lambda: (0, 0)),
            pl.BlockSpec((1, NUM_CLASSES), lambda: (0, 0)),
        ],
        out_specs=pl.BlockSpec((BATCH, NUM_CLASSES), lambda: (0, 0)),
        out_shape=jax.ShapeDtypeStruct((BATCH, NUM_CLASSES), jnp.float32),
    )(hcls, lw, lb, w, b)


_CH = 128
_NCHUNK = NA // _CH
_NW = 32


def _chunk_rows(wid):
    return [wid, wid + _NW]


@functools.cache
def _sc_dispatch_fn():
    mesh = plsc.VectorSubcoreMesh(core_axis_name="c", subcore_axis_name="s")

    @functools.partial(
        pl.kernel, mesh=mesh,
        out_type=jax.ShapeDtypeStruct((MPAD, EMBED), jnp.float32),
        scratch_types=[
            pltpu.VMEM((_CH,), jnp.int32),
            pltpu.VMEM((_CH, EMBED), jnp.float32),
            pltpu.SemaphoreType.DMA,
        ],
    )
    def k(xln_hbm, pp_hbm, xs_hbm, idx_v, rows_v, sem):
        wid = lax.axis_index("s") * 2 + lax.axis_index("c")
        for j in _chunk_rows(wid):
            @pl.when(j < _NCHUNK)
            def _():
                tb = jnp.where(j < _NCHUNK // 2, j * _CH, j * _CH - N)
                pltpu.sync_copy(pp_hbm.at[j], idx_v)
                pltpu.sync_copy(xln_hbm.at[pl.ds(tb, _CH)], rows_v)
                pltpu.async_copy(rows_v, xs_hbm.at[idx_v], sem).wait()

    return k


@functools.cache
def _sc_collect_fn():
    mesh = plsc.VectorSubcoreMesh(core_axis_name="c", subcore_axis_name="s")

    @functools.partial(
        pl.kernel, mesh=mesh,
        out_type=jax.ShapeDtypeStruct((NA, EMBED), jnp.float32),
        scratch_types=[
            pltpu.VMEM((_CH,), jnp.int32),
            pltpu.VMEM((_CH, EMBED), jnp.float32),
            pltpu.SemaphoreType.DMA,
        ],
    )
    def k(eo_hbm, pp_hbm, g_hbm, idx_v, rows_v, sem):
        wid = lax.axis_index("s") * 2 + lax.axis_index("c")
        for j in _chunk_rows(wid):
            @pl.when(j < _NCHUNK)
            def _():
                pltpu.sync_copy(pp_hbm.at[j], idx_v)
                pltpu.async_copy(eo_hbm.at[idx_v], rows_v, sem).wait()
                pltpu.sync_copy(rows_v, g_hbm.at[pl.ds(j * _CH, _CH)])

    return k


def _sc_dispatch(xln, pp2):
    return _sc_dispatch_fn()(xln, pp2)


def _sc_collect(eo, pp2):
    return _sc_collect_fn()(eo, pp2)


_RTILE = 256


def _rank_body(e_ref, r_ref, cnt_ref, carry):
    g = pl.program_id(0)

    @pl.when(g == 0)
    def _():
        carry[...] = jnp.zeros_like(carry)

    e = e_ref[...]
    iot = lax.broadcasted_iota(jnp.int32, (_RTILE, NUM_EXPERT), 1)
    oh = (e == iot).astype(jnp.float32)
    row = lax.broadcasted_iota(jnp.int32, (_RTILE, _RTILE), 0)
    col = lax.broadcasted_iota(jnp.int32, (_RTILE, _RTILE), 1)
    tri = (row > col).astype(jnp.float32)
    within = _mm_hi(tri, oh)
    r = jnp.sum(oh * (carry[...] + within), axis=1, keepdims=True)
    r_ref[...] = r
    carry[...] = carry[...] + jnp.sum(oh, axis=0, keepdims=True)
    cnt_ref[...] = carry[...]


def _rank_scan(ef):
    r, cnt = pl.pallas_call(
        _rank_body,
        grid=(NA // _RTILE,),
        in_specs=[pl.BlockSpec((_RTILE, 1), lambda i: (i, 0))],
        out_specs=[
            pl.BlockSpec((_RTILE, 1), lambda i: (i, 0)),
            pl.BlockSpec((1, NUM_EXPERT), lambda i: (0, 0)),
        ],
        out_shape=[
            jax.ShapeDtypeStruct((NA, 1), jnp.float32),
            jax.ShapeDtypeStruct((1, NUM_EXPERT), jnp.float32),
        ],
        scratch_shapes=[pltpu.VMEM((1, NUM_EXPERT), jnp.float32)],
    )(ef.reshape(NA, 1))
    return r.reshape(NA).astype(jnp.int32), cnt.reshape(NUM_EXPERT).astype(jnp.int32)


def _route(i01):
    ef = jnp.concatenate([i01[:, 0], i01[:, 1]])
    r, counts = _rank_scan(ef)
    cpad = ((counts + TILE - 1) // TILE) * TILE
    pad_off = jnp.concatenate([jnp.zeros((1,), cpad.dtype), jnp.cumsum(cpad)[:-1]])
    pp = (pad_off[ef] + r).astype(jnp.int32)
    tile_start = jnp.arange(NTILES) * TILE
    eid = jnp.clip(
        jnp.searchsorted(pad_off, tile_start, side="right") - 1,
        0, NUM_EXPERT - 1).astype(jnp.int32)
    return pp.reshape(_NCHUNK, _CH), eid


def kernel(x, params):
    p = params
    emb = _embed(x, p["patch_w"], p["patch_b"].reshape(1, EMBED))
    emb = emb.reshape(BATCH, NUM_PATCHES, EMBED)
    cls = jnp.broadcast_to(p["cls"], (BATCH, 1, EMBED))
    h = jnp.concatenate([cls, emb], axis=1) + p["pos"]
    h = jnp.pad(h, ((0, 0), (0, TP - T), (0, 0))).reshape(N, EMBED)

    for blk in p["blocks"]:
        r1 = lambda a: a.reshape(1, -1)
        qkv = _ln_qkv(h, r1(blk["ln1_w"]), r1(blk["ln1_b"]),
                      blk["qkv_w"], r1(blk["qkv_b"]))
        ao = _attn(qkv)
        if "w1" in blk:
            h2, xn, i01, s01 = _proj_gate(
                ao, h, blk["proj_w"], r1(blk["proj_b"]),
                r1(blk["ln2_w"]), r1(blk["ln2_b"]),
                blk["gate_w"], r1(blk["gate_b"]))
            pp2, eid = _route(i01)
            x_sorted = _sc_dispatch(xn, pp2)
            eo = _ggemm(x_sorted, blk["w1"], blk["b1"].reshape(NUM_EXPERT, 1, HID),
                        blk["w2"], blk["b2"].reshape(NUM_EXPERT, 1, EMBED), eid)
            g = _sc_collect(eo, pp2)
            h = _combine(h2, g, s01)
        else:
            h = _proj_mlp(ao, h, blk["proj_w"], r1(blk["proj_b"]),
                          r1(blk["ln2_w"]), r1(blk["ln2_b"]),
                          blk["fc1_w"], r1(blk["fc1_b"]),
                          blk["fc2_w"], r1(blk["fc2_b"]))

    hcls = h.reshape(BATCH, TP, EMBED)[:, 0, :]
    return _head(hcls, r1(p["norm_w"]), r1(p["norm_b"]),
                 p["head_w"], r1(p["head_b"]))
